# Initial kernel scaffold; baseline (speedup 1.0000x reference)
import jax
import jax.numpy as jnp
from jax.experimental import pallas as pl


def kernel(features, edge_index, e_feat, W_self0, W_neigh0, b0, W_self1, W_neigh1, b1, W_self2, W_neigh2, b2, W_ro, b_ro):
    raise NotImplementedError("write your pallas kernel here")


# speedup vs baseline: 5.2878x; 5.2878x over previous
import functools

import jax
import jax.numpy as jnp
from jax import lax
from jax.experimental import pallas as pl
from jax.experimental.pallas import tpu as pltpu
from jax.experimental.pallas import tpu_sc as plsc

NC = 2
NS = 16
NW = NC * NS
CH = 128


def _make_sc_pass(NP, EPW, layer0):
    nch = EPW // CH
    rows_per_tile = NP // NS
    ZB = 1280
    nz = rows_per_tile // ZB
    mesh = plsc.VectorSubcoreMesh(core_axis_name="c", subcore_axis_name="s")

    @functools.partial(
        pl.kernel,
        out_type=jax.ShapeDtypeStruct((NC, NP, 16), jnp.float32),
        mesh=mesh,
        compiler_params=pltpu.CompilerParams(
            needs_layout_passes=False, use_tc_tiling_on_sc=False),
        scratch_types=[
            pltpu.VMEM((CH,), jnp.int32),
            pltpu.VMEM((CH,), jnp.int32),
            pltpu.VMEM((CH,), jnp.float32),
            pltpu.VMEM((CH, 16), jnp.float32),
            pltpu.VMEM((ZB, 16), jnp.float32),
            pltpu.VMEM_SHARED((NP, 16), jnp.float32),
            pltpu.SemaphoreType.DMA,
        ],
    )
    def sc_pass(table, srcs, dsts, ws, out, sidx_v, didx_v, w_v, rows_v,
                zbuf_v, acc, sem):
        c = lax.axis_index("c")
        s = lax.axis_index("s")
        wid = c * NS + s
        my_rows = s * rows_per_tile

        zero16 = jnp.zeros((16,), jnp.float32)

        def zb_body(i, carry):
            zbuf_v[i] = zero16
            return carry

        lax.fori_loop(0, ZB, zb_body, 0)
        for t in range(nz):
            pltpu.sync_copy(zbuf_v, acc.at[pl.ds(my_rows + t * ZB, ZB)])
        plsc.subcore_barrier()

        lanes = lax.broadcasted_iota(jnp.int32, (16,), 0)
        if layer0:
            c1 = jnp.where(lanes < 4, 1.0, 0.0).astype(jnp.float32)
            c2 = jnp.where(lanes == 4, 1.0, 0.0).astype(jnp.float32)

        base = wid * EPW

        def chunk(j, carry):
            off = base + j * CH
            pltpu.sync_copy(srcs.at[pl.ds(off, CH)], sidx_v)
            pltpu.sync_copy(dsts.at[pl.ds(off, CH)], didx_v)
            pltpu.sync_copy(ws.at[pl.ds(off, CH)], w_v)
            pltpu.async_copy(table.at[sidx_v], rows_v, sem).wait()

            def edge(e, ecarry):
                wsplat = plsc.load_gather(w_v, [jnp.broadcast_to(e, (16,))])
                if layer0:
                    m = wsplat * c1 + c2
                else:
                    m = wsplat
                rows_v[e] = rows_v[e] * m
                return ecarry

            lax.fori_loop(0, CH, edge, 0)
            pltpu.sync_copy(rows_v, acc.at[didx_v], add=True)
            return carry

        lax.fori_loop(0, nch, chunk, 0)
        plsc.subcore_barrier()

        for t in range(nz):
            pltpu.sync_copy(acc.at[pl.ds(my_rows + t * ZB, ZB)], zbuf_v)
            pltpu.sync_copy(zbuf_v, out.at[c, pl.ds(my_rows + t * ZB, ZB)])

    return sc_pass


_TCB = 1024


def _tc1_body(f_ref, p_ref, wsf_ref, wnf_ref, wno_ref, beff_ref,
              h1_ref, invd_ref):
    agg = p_ref[0] + p_ref[1]
    sumw = agg[:, 3:4]
    deg = agg[:, 4:5]
    invd = 1.0 / jnp.maximum(deg, 1.0)
    aggf = agg[:, 0:3]
    f = f_ref[...]
    pre = jnp.dot(aggf, wnf_ref[...], preferred_element_type=jnp.float32)
    pre = pre + sumw * wno_ref[...]
    act = jnp.dot(f, wsf_ref[...], preferred_element_type=jnp.float32)
    act = act + invd * pre + beff_ref[...]
    h1_ref[...] = jax.nn.sigmoid(act)
    invd_ref[...] = invd


def _tc_mid_body(h_ref, p_ref, invd_ref, ws_ref, wn_ref, b_ref, out_ref):
    agg = (p_ref[0] + p_ref[1]) * invd_ref[...]
    act = jnp.dot(h_ref[...], ws_ref[...], preferred_element_type=jnp.float32)
    act = act + jnp.dot(agg, wn_ref[...], preferred_element_type=jnp.float32)
    out_ref[...] = jax.nn.sigmoid(act + b_ref[...])


def _tc_last_body(h_ref, p_ref, invd_ref, f_ref, ws_ref, wn_ref, b_ref,
                  wrof_ref, wroh_ref, bro_ref, out_ref):
    agg = (p_ref[0] + p_ref[1]) * invd_ref[...]
    h3 = jnp.dot(h_ref[...], ws_ref[...], preferred_element_type=jnp.float32)
    h3 = h3 + jnp.dot(agg, wn_ref[...], preferred_element_type=jnp.float32)
    h3 = h3 + b_ref[...]
    out = jnp.dot(f_ref[...], wrof_ref[...], preferred_element_type=jnp.float32)
    out = out + jnp.dot(h3, wroh_ref[...], preferred_element_type=jnp.float32)
    out_ref[...] = out + bro_ref[...]


def _row_spec(cols):
    return pl.BlockSpec((_TCB, cols), lambda i: (i, 0))


def _part_spec():
    return pl.BlockSpec((NC, _TCB, 16), lambda i: (0, i, 0))


def _full_spec(r, c):
    return pl.BlockSpec((r, c), lambda i: (0, 0))


def kernel(features, edge_index, e_feat,
           W_self0, W_neigh0, b0,
           W_self1, W_neigh1, b1,
           W_self2, W_neigh2, b2,
           W_ro, b_ro):
    N = features.shape[0]
    E = edge_index.shape[1]
    f32 = jnp.float32

    NP = -(-N // 102400) * 102400
    EPW = ((E + NW * CH - 1) // (NW * CH)) * CH
    EP = EPW * NW
    pad = EP - E
    src = edge_index[0]
    dst = edge_index[1]
    w = e_feat[:, 0]
    if pad:
        src = jnp.concatenate([src, jnp.zeros((pad,), jnp.int32)])
        dst = jnp.concatenate([dst, N + (jnp.arange(pad, dtype=jnp.int32) % 8)])
        w = jnp.concatenate([w, jnp.zeros((pad,), f32)])

    npad = NP - N
    fpad = jnp.concatenate([features, jnp.zeros((npad, 3), f32)])
    ones = jnp.ones((N, 1), f32)
    t0 = jnp.concatenate([features, ones, ones, jnp.zeros((N, 11), f32)], axis=1)
    t0 = jnp.concatenate([t0, jnp.zeros((npad, 16), f32)])

    Wsf0 = W_self0[:3]
    beff0 = (b0 + W_self0[3:].sum(0))[None, :]
    Wnf0 = W_neigh0[:3]
    wno0 = W_neigh0[3:].sum(0)[None, :]

    sc0 = _make_sc_pass(NP, EPW, layer0=True)
    sc = _make_sc_pass(NP, EPW, layer0=False)

    grid = (NP // _TCB,)

    p0 = sc0(t0, src, dst, w)
    h1, invd = pl.pallas_call(
        _tc1_body,
        grid=grid,
        in_specs=[_row_spec(3), _part_spec(), _full_spec(3, 16),
                  _full_spec(3, 16), _full_spec(1, 16), _full_spec(1, 16)],
        out_specs=[_row_spec(16), _row_spec(1)],
        out_shape=[jax.ShapeDtypeStruct((NP, 16), f32),
                   jax.ShapeDtypeStruct((NP, 1), f32)],
    )(fpad, p0, Wsf0, Wnf0, wno0, beff0)

    p1 = sc(h1, src, dst, w)
    h2 = pl.pallas_call(
        _tc_mid_body,
        grid=grid,
        in_specs=[_row_spec(16), _part_spec(), _row_spec(1),
                  _full_spec(16, 16), _full_spec(16, 16), _full_spec(1, 16)],
        out_specs=_row_spec(16),
        out_shape=jax.ShapeDtypeStruct((NP, 16), f32),
    )(h1, p1, invd, W_self1, W_neigh1, b1[None, :])

    p2 = sc(h2, src, dst, w)
    out = pl.pallas_call(
        _tc_last_body,
        grid=grid,
        in_specs=[_row_spec(16), _part_spec(), _row_spec(1), _row_spec(3),
                  _full_spec(16, 16), _full_spec(16, 16), _full_spec(1, 16),
                  _full_spec(3, 1), _full_spec(16, 1), _full_spec(1, 1)],
        out_specs=_row_spec(1),
        out_shape=jax.ShapeDtypeStruct((NP, 1), f32),
    )(h2, p2, invd, fpad, W_self2, W_neigh2, b2[None, :],
      W_ro[:3], W_ro[3:], b_ro[None, :])

    return out[:N]


# speedup vs baseline: 9.5580x; 1.8076x over previous
import functools

import jax
import jax.numpy as jnp
from jax import lax
from jax.experimental import pallas as pl
from jax.experimental.pallas import tpu as pltpu
from jax.experimental.pallas import tpu_sc as plsc

NC = 2
NS = 16
NW = NC * NS
CH = 128


def _make_sc_pass(NP, EPW, layer0):
    nch = EPW // CH
    NSB = 10
    sb_ch = nch // NSB
    rows_per_tile = NP // NS
    ZB = 256
    nz = rows_per_tile // ZB
    mesh = plsc.VectorSubcoreMesh(core_axis_name="c", subcore_axis_name="s")

    @functools.partial(
        pl.kernel,
        out_type=jax.ShapeDtypeStruct((NC, NP, 16), jnp.float32),
        mesh=mesh,
        compiler_params=pltpu.CompilerParams(
            needs_layout_passes=False, use_tc_tiling_on_sc=False),
        scratch_types=[
            pltpu.VMEM((sb_ch, CH), jnp.int32),
            pltpu.VMEM((sb_ch, CH), jnp.int32),
            pltpu.VMEM((sb_ch, CH), jnp.float32),
            pltpu.VMEM((2, CH, 16), jnp.float32),
            pltpu.VMEM((ZB, 16), jnp.float32),
            pltpu.VMEM_SHARED((NP, 16), jnp.float32),
            pltpu.SemaphoreType.DMA,
            pltpu.SemaphoreType.DMA,
        ],
    )
    def sc_pass(table, srcs, dsts, ws, out, srcb, dstb, wb, rows_v,
                zbuf_v, acc, ssem, gsem):
        c = lax.axis_index("c")
        s = lax.axis_index("s")
        wid = c * NS + s
        my_rows = s * rows_per_tile

        zero16 = jnp.zeros((16,), jnp.float32)

        def zb_body(i, carry):
            zbuf_v[i] = zero16
            return carry

        lax.fori_loop(0, ZB, zb_body, 0)
        for t in range(nz):
            pltpu.sync_copy(zbuf_v, acc.at[pl.ds(my_rows + t * ZB, ZB)])
        plsc.subcore_barrier()

        lanes = lax.broadcasted_iota(jnp.int32, (16,), 0)

        def gather_desc(j, b):
            return pltpu.make_async_copy(table.at[srcb.at[j]],
                                         rows_v.at[b], gsem)

        for sb in range(NSB):
            roff = wid * nch + sb * sb_ch
            d1 = pltpu.make_async_copy(srcs.at[pl.ds(roff, sb_ch)], srcb, ssem)
            d2 = pltpu.make_async_copy(dsts.at[pl.ds(roff, sb_ch)], dstb, ssem)
            d3 = pltpu.make_async_copy(ws.at[pl.ds(roff, sb_ch)], wb, ssem)
            d1.start()
            d2.start()
            d3.start()
            d1.wait()
            d2.wait()
            d3.wait()

            gather_desc(0, 0).start()

            ndim = 4 if layer0 else 16

            def chunk(j, carry):
                b = jnp.bitwise_and(j, 1)

                @pl.when(j < sb_ch - 1)
                def _():
                    gather_desc(j + 1, 1 - b).start()

                gather_desc(j, b).wait()

                bvec = jnp.broadcast_to(b, (16,))
                for g in range(CH // 16):
                    wvec = wb[j, pl.ds(g * 16, 16)]
                    evec = lanes + (g * 16)
                    for d in range(ndim):
                        dvec = jnp.broadcast_to(jnp.int32(d), (16,))
                        vals = plsc.load_gather(rows_v, [bvec, evec, dvec])
                        plsc.store_scatter(rows_v, [bvec, evec, dvec],
                                           vals * wvec)

                pltpu.sync_copy(rows_v.at[b], acc.at[dstb.at[j]], add=True)
                return carry

            lax.fori_loop(0, sb_ch, chunk, 0)

        plsc.subcore_barrier()

        for t in range(nz):
            pltpu.sync_copy(acc.at[pl.ds(my_rows + t * ZB, ZB)], zbuf_v)
            pltpu.sync_copy(zbuf_v, out.at[c, pl.ds(my_rows + t * ZB, ZB)])

    return sc_pass


_TCB = 1024


def _tc1_body(f_ref, p_ref, wsf_ref, wnf_ref, wno_ref, beff_ref,
              h1_ref, invd_ref):
    agg = p_ref[0] + p_ref[1]
    sumw = agg[:, 3:4]
    deg = agg[:, 4:5]
    invd = 1.0 / jnp.maximum(deg, 1.0)
    aggf = agg[:, 0:3]
    f = f_ref[...]
    pre = jnp.dot(aggf, wnf_ref[...], preferred_element_type=jnp.float32)
    pre = pre + sumw * wno_ref[...]
    act = jnp.dot(f, wsf_ref[...], preferred_element_type=jnp.float32)
    act = act + invd * pre + beff_ref[...]
    h1_ref[...] = jax.nn.sigmoid(act)
    invd_ref[...] = invd


def _tc_mid_body(h_ref, p_ref, invd_ref, ws_ref, wn_ref, b_ref, out_ref):
    agg = (p_ref[0] + p_ref[1]) * invd_ref[...]
    act = jnp.dot(h_ref[...], ws_ref[...], preferred_element_type=jnp.float32)
    act = act + jnp.dot(agg, wn_ref[...], preferred_element_type=jnp.float32)
    out_ref[...] = jax.nn.sigmoid(act + b_ref[...])


def _tc_last_body(h_ref, p_ref, invd_ref, f_ref, ws_ref, wn_ref, b_ref,
                  wrof_ref, wroh_ref, bro_ref, out_ref):
    agg = (p_ref[0] + p_ref[1]) * invd_ref[...]
    h3 = jnp.dot(h_ref[...], ws_ref[...], preferred_element_type=jnp.float32)
    h3 = h3 + jnp.dot(agg, wn_ref[...], preferred_element_type=jnp.float32)
    h3 = h3 + b_ref[...]
    out = jnp.dot(f_ref[...], wrof_ref[...], preferred_element_type=jnp.float32)
    out = out + jnp.dot(h3, wroh_ref[...], preferred_element_type=jnp.float32)
    out_ref[...] = out + bro_ref[...]


def _row_spec(cols):
    return pl.BlockSpec((_TCB, cols), lambda i: (i, 0))


def _part_spec():
    return pl.BlockSpec((NC, _TCB, 16), lambda i: (0, i, 0))


def _full_spec(r, c):
    return pl.BlockSpec((r, c), lambda i: (0, 0))


def kernel(features, edge_index, e_feat,
           W_self0, W_neigh0, b0,
           W_self1, W_neigh1, b1,
           W_self2, W_neigh2, b2,
           W_ro, b_ro):
    N = features.shape[0]
    E = edge_index.shape[1]
    f32 = jnp.float32

    NP = -(-N // 102400) * 102400
    EPW = ((E + NW * CH * 16 - 1) // (NW * CH * 16)) * CH * 16
    EP = EPW * NW
    pad = EP - E
    src = edge_index[0]
    dst = edge_index[1]
    w = e_feat[:, 0]
    if pad:
        src = jnp.concatenate([src, jnp.zeros((pad,), jnp.int32)])
        dst = jnp.concatenate(
            [dst, N + (jnp.arange(pad, dtype=jnp.int32) % (NP - N))])
        w = jnp.concatenate([w, jnp.zeros((pad,), f32)])
    src = src.reshape(EP // CH, CH)
    dst = dst.reshape(EP // CH, CH)
    w = w.reshape(EP // CH, CH)

    npad = NP - N
    fpad = jnp.concatenate([features, jnp.zeros((npad, 3), f32)])
    ones = jnp.ones((N, 1), f32)
    t0 = jnp.concatenate([features, ones, ones, jnp.zeros((N, 11), f32)], axis=1)
    t0 = jnp.concatenate([t0, jnp.zeros((npad, 16), f32)])

    Wsf0 = W_self0[:3]
    beff0 = (b0 + W_self0[3:].sum(0))[None, :]
    Wnf0 = W_neigh0[:3]
    wno0 = W_neigh0[3:].sum(0)[None, :]

    sc0 = _make_sc_pass(NP, EPW, layer0=True)
    sc = _make_sc_pass(NP, EPW, layer0=False)

    grid = (NP // _TCB,)

    p0 = sc0(t0, src, dst, w)
    h1, invd = pl.pallas_call(
        _tc1_body,
        grid=grid,
        in_specs=[_row_spec(3), _part_spec(), _full_spec(3, 16),
                  _full_spec(3, 16), _full_spec(1, 16), _full_spec(1, 16)],
        out_specs=[_row_spec(16), _row_spec(1)],
        out_shape=[jax.ShapeDtypeStruct((NP, 16), f32),
                   jax.ShapeDtypeStruct((NP, 1), f32)],
    )(fpad, p0, Wsf0, Wnf0, wno0, beff0)

    p1 = sc(h1, src, dst, w)
    h2 = pl.pallas_call(
        _tc_mid_body,
        grid=grid,
        in_specs=[_row_spec(16), _part_spec(), _row_spec(1),
                  _full_spec(16, 16), _full_spec(16, 16), _full_spec(1, 16)],
        out_specs=_row_spec(16),
        out_shape=jax.ShapeDtypeStruct((NP, 16), f32),
    )(h1, p1, invd, W_self1, W_neigh1, b1[None, :])

    p2 = sc(h2, src, dst, w)
    out = pl.pallas_call(
        _tc_last_body,
        grid=grid,
        in_specs=[_row_spec(16), _part_spec(), _row_spec(1), _row_spec(3),
                  _full_spec(16, 16), _full_spec(16, 16), _full_spec(1, 16),
                  _full_spec(3, 1), _full_spec(16, 1), _full_spec(1, 1)],
        out_specs=_row_spec(1),
        out_shape=jax.ShapeDtypeStruct((NP, 1), f32),
    )(h2, p2, invd, fpad, W_self2, W_neigh2, b2[None, :],
      W_ro[:3], W_ro[3:], b_ro[None, :])

    return out[:N]
